# Initial kernel scaffold; baseline (speedup 1.0000x reference)
import jax
import jax.numpy as jnp
from jax.experimental import pallas as pl


def kernel(features_drug, features_dis, params, edge_d0, edge_d1, edge_s0, edge_s1, type_mask):
    raise NotImplementedError("write your pallas kernel here")


# speedup vs baseline: 1.0545x; 1.0545x over previous
import functools

import jax
import jax.numpy as jnp
from jax.experimental import pallas as pl
from jax.experimental.pallas import tpu as pltpu

N_TYPE = 50000
E = 800000
D_FEAT = 128
HID = 32
H = 4
DH = 32
SEM_HID = 128
OUT = 32


def _gat(h, edge, W, al, ar, b):
    N = h.shape[0]
    feat = (h @ W.T).reshape(N, H, DH)
    el = (feat * al[None]).sum(-1)
    er = (feat * ar[None]).sum(-1)
    src, dst = edge[0], edge[1]
    e = jax.nn.leaky_relu(el[src] + er[dst], 0.2)
    ex = jnp.exp(e)
    den = jax.ops.segment_sum(ex, dst, num_segments=N)
    out = jax.ops.segment_sum(feat[src] * ex[..., None], dst, num_segments=N)
    out = out / jnp.maximum(den, 1e-9)[..., None]
    out = out + b.reshape(1, H, DH)
    return jax.nn.elu(out).reshape(N, H * DH)


def _semantic(z, W1, b1, W2):
    w = (jnp.tanh(z @ W1.T + b1) @ W2.T).mean(0)
    beta = jax.nn.softmax(w, axis=0)
    return (beta[None] * z).sum(1)


def _han(p, L, feats, edges):
    embs = []
    for m in (0, 1):
        g = _gat(feats, edges[m], p[f'gat{L}{m}_W'], p[f'gat{L}{m}_al'],
                 p[f'gat{L}{m}_ar'], p[f'gat{L}{m}_b'])
        embs.append(g)
    z = jnp.stack(embs, axis=1)
    hsem = _semantic(z, p[f'sem{L}_W1'], p[f'sem{L}_b1'], p[f'sem{L}_W2'])
    return hsem @ p[f'han{L}_W'].T + p[f'han{L}_b']


def _att_block_kernel(q_ref, k_ref, v_ref, qw, qb, kw, kb, vw, vb, ow, ob, o_ref):
    scale = jnp.sqrt(jnp.float32(OUT))
    Q = q_ref[...] @ qw[...].T + qb[...]
    K = k_ref[...] @ kw[...].T + kb[...]
    V = v_ref[...] @ vw[...].T + vb[...]
    B = Q.shape[0]
    en = Q[:, :, None] * K[:, None, :] / scale
    en = en - jnp.max(en, axis=-1, keepdims=True)
    ex = jnp.exp(en)
    attn = ex / jnp.sum(ex, axis=-1, keepdims=True)
    wm = jnp.einsum('bij,bj->bi', attn, V)
    o_ref[...] = wm @ ow[...].T + ob[...]


def _att(p, q, k, v):
    B = q.shape[0]
    BN = 1000
    grid = (B // BN,)
    spec_row = pl.BlockSpec((BN, OUT), lambda i: (i, 0))
    spec_w = pl.BlockSpec((OUT, OUT), lambda i: (0, 0))
    spec_b = pl.BlockSpec((OUT,), lambda i: (0,))
    return pl.pallas_call(
        _att_block_kernel,
        grid=grid,
        in_specs=[spec_row, spec_row, spec_row,
                  spec_w, spec_b, spec_w, spec_b, spec_w, spec_b, spec_w, spec_b],
        out_specs=spec_row,
        out_shape=jax.ShapeDtypeStruct((B, OUT), jnp.float32),
    )(q, k, v,
      p['att_q_W'], p['att_q_b'], p['att_k_W'], p['att_k_b'],
      p['att_v_W'], p['att_v_b'], p['att_o_W'], p['att_o_b'])


def kernel(features_drug, features_dis, params, edge_d0, edge_d1, edge_s0, edge_s1, type_mask):
    p = params
    tf0 = features_drug @ p['fc0_W'].T + p['fc0_b']
    tf1 = features_dis @ p['fc1_W'].T + p['fc1_b']
    drug_h = _han(p, 1, tf0, [edge_d0, edge_d1])
    dis_h = _han(p, 2, tf1, [edge_s0, edge_s1])
    drug_o = _att(p, drug_h, dis_h, dis_h)
    dis_o = _att(p, dis_h, drug_o, drug_o)
    return drug_o, dis_o


# speedup vs baseline: 32.3280x; 30.6579x over previous
import functools

import jax
import jax.numpy as jnp
from jax import lax
from jax.experimental import pallas as pl
from jax.experimental.pallas import tpu as pltpu
from jax.experimental.pallas import tpu_sc as plsc

N = 50000
E = 800000
D_FEAT = 128
HID = 32
H = 4
DH = 32
OUT = 32
EL = 16

NT = 16
NSC = 2
ROWS_PER_TILE = N // NT
CA = 1000
CB = 2000


def _proj_body(fd, fs, fc0w, fc1w, w0, w1, w2, w3,
               alz0, alz1, alz2, alz3, arz0, arz1, arz2, arz3,
               feat0, feat1, feat2, feat3,
               el0, el1, el2, el3, er0, er1, er2, er3):
    tf0 = jnp.dot(fd[...], fc0w[...], preferred_element_type=jnp.float32)
    tf1 = jnp.dot(fs[...], fc1w[...], preferred_element_type=jnp.float32)
    for tf, w, alz, arz, fo, elo, ero in (
            (tf0, w0, alz0, arz0, feat0, el0, er0),
            (tf0, w1, alz1, arz1, feat1, el1, er1),
            (tf1, w2, alz2, arz2, feat2, el2, er2),
            (tf1, w3, alz3, arz3, feat3, el3, er3)):
        f = jnp.dot(tf, w[...], preferred_element_type=jnp.float32)
        fo[...] = f
        elo[...] = jnp.dot(f, alz[...], preferred_element_type=jnp.float32)
        ero[...] = jnp.dot(f, arz[...], preferred_element_type=jnp.float32)


def _k1(fd, fs, fc0w, fc1w, ws, alzs, arzs):
    BN = 1000
    grid = (N // BN,)
    row = pl.BlockSpec((BN, D_FEAT), lambda i: (i, 0))
    w32 = pl.BlockSpec((D_FEAT, HID), lambda i: (0, 0))
    w128 = pl.BlockSpec((HID, H * DH), lambda i: (0, 0))
    wz = pl.BlockSpec((H * DH, EL), lambda i: (0, 0))
    orow = pl.BlockSpec((BN, H * DH), lambda i: (i, 0))
    oel = pl.BlockSpec((BN, EL), lambda i: (i, 0))
    out_shape = ([jax.ShapeDtypeStruct((N, H * DH), jnp.float32)] * 4
                 + [jax.ShapeDtypeStruct((N, EL), jnp.float32)] * 8)
    return pl.pallas_call(
        _proj_body,
        grid=grid,
        in_specs=[row, row, w32, w32] + [w128] * 4 + [wz] * 8,
        out_specs=[orow] * 4 + [oel] * 8,
        out_shape=out_shape,
    )(fd, fs, fc0w, fc1w, *ws, *alzs, *arzs)


def _edge_a(src, dst, el, er, z4):
    mesh = plsc.VectorSubcoreMesh(core_axis_name="c", subcore_axis_name="s")
    epsc = E // NSC
    nchunk = epsc // CA // NT

    @functools.partial(
        pl.kernel,
        out_type=[jax.ShapeDtypeStruct((E, EL), jnp.float32),
                  jax.ShapeDtypeStruct((NSC, N, EL), jnp.float32)],
        mesh=mesh,
        compiler_params=pltpu.CompilerParams(use_tc_tiling_on_sc=False, needs_layout_passes=False),
        scratch_types=[
            pltpu.VMEM((CA,), jnp.int32),
            pltpu.VMEM((CA,), jnp.int32),
            pltpu.VMEM((CA, EL), jnp.float32),
            pltpu.VMEM((CA, EL), jnp.float32),
            pltpu.VMEM((CA, EL), jnp.float32),
            pltpu.VMEM_SHARED((N, EL), jnp.float32),
        ],
    )
    def k(src_h, dst_h, el_h, er_h, z4_h, ex_h, den_h,
          idxs, idxd, elr, err, exb, den_sh):
        cid = lax.axis_index("c")
        sid = lax.axis_index("s")

        @pl.when(sid == 0)
        def _zero():
            pltpu.sync_copy(z4_h, den_sh)

        plsc.subcore_barrier()

        def chunk(kk, _):
            base = cid * epsc + (kk * NT + sid) * CA
            pltpu.sync_copy(src_h.at[pl.ds(base, CA)], idxs)
            pltpu.sync_copy(dst_h.at[pl.ds(base, CA)], idxd)
            pltpu.sync_copy(el_h.at[idxs], elr)
            pltpu.sync_copy(er_h.at[idxd], err)

            def cb(e, _):
                a = elr[e, :] + err[e, :]
                a = jnp.where(a >= 0.0, a, 0.2 * a)
                exb[e, :] = jnp.exp(a)
                return 0

            lax.fori_loop(0, CA, cb, 0)
            pltpu.sync_copy(exb, ex_h.at[pl.ds(base, CA)])
            pltpu.sync_copy(exb, den_sh.at[idxd], add=True)
            return 0

        lax.fori_loop(0, nchunk, chunk, 0)
        plsc.subcore_barrier()

        @pl.when(sid == 0)
        def _out():
            pltpu.sync_copy(den_sh, den_h.at[cid])

    return k(src, dst, el, er, z4)


def _ext_body(ex_ref, out_ref):
    x = ex_ref[...]
    out_ref[...] = x.T[0:8, :]


def _ex_transpose(ex):
    BN = 6400
    grid = (E // BN,)
    return pl.pallas_call(
        _ext_body,
        grid=grid,
        in_specs=[pl.BlockSpec((BN, EL), lambda i: (i, 0))],
        out_specs=pl.BlockSpec((8, BN), lambda i: (0, i)),
        out_shape=jax.ShapeDtypeStruct((8, E), jnp.float32),
    )(ex)


def _edge_b(src, dst, ex, featv, z4):
    mesh = plsc.VectorSubcoreMesh(core_axis_name="c", subcore_axis_name="s")
    nchunk = E // CB // NT

    @functools.partial(
        pl.kernel,
        out_type=jax.ShapeDtypeStruct((N, H, 2, EL), jnp.float32),
        mesh=mesh,
        compiler_params=pltpu.CompilerParams(use_tc_tiling_on_sc=False, needs_layout_passes=False),
        scratch_types=[
            pltpu.VMEM((CB,), jnp.int32),
            pltpu.VMEM((CB,), jnp.int32),
            pltpu.VMEM((CB,), jnp.int32),
            pltpu.VMEM((CB,), jnp.float32),
            pltpu.VMEM((CB, EL), jnp.float32),
            pltpu.VMEM_SHARED((N, EL), jnp.float32),
        ],
    )
    def k(src_h, dst_h, ex_h, featv_h, z4_h, out_h,
          idxs, idxd, idxg, exr, rows, acc_sh):
        cid = lax.axis_index("c")
        sid = lax.axis_index("s")

        for hh in range(2):
            h = cid * 2 + hh
            for half in range(2):
                r0 = sid * ROWS_PER_TILE
                pltpu.sync_copy(z4_h.at[pl.ds(r0, ROWS_PER_TILE)],
                                acc_sh.at[pl.ds(r0, ROWS_PER_TILE)])
                plsc.subcore_barrier()

                def chunk(kk, _):
                    base = (kk * NT + sid) * CB
                    pltpu.sync_copy(src_h.at[pl.ds(base, CB)], idxs)
                    pltpu.sync_copy(dst_h.at[pl.ds(base, CB)], idxd)
                    pltpu.sync_copy(ex_h.at[h, pl.ds(base, CB)], exr)
                    sub = h * 2 + half

                    def ib(i, _):
                        v = idxs[pl.ds(i * 16, 16)]
                        idxg[pl.ds(i * 16, 16)] = v * (2 * H) + sub
                        return 0

                    lax.fori_loop(0, CB // 16, ib, 0)
                    pltpu.sync_copy(featv_h.at[idxg], rows)

                    def mul(e, _):
                        xv = plsc.load_gather(
                            exr, [jnp.full((16,), e, jnp.int32)])
                        rows[e, :] = rows[e, :] * xv
                        return 0

                    lax.fori_loop(0, CB, mul, 0)
                    pltpu.sync_copy(rows, acc_sh.at[idxd], add=True)
                    return 0

                lax.fori_loop(0, nchunk, chunk, 0)
                plsc.subcore_barrier()
                pltpu.sync_copy(acc_sh.at[pl.ds(r0, ROWS_PER_TILE)],
                                out_h.at[pl.ds(r0, ROWS_PER_TILE), h, half])
                plsc.subcore_barrier()

    return k(src, dst, ex, featv, z4)


def _post_body(o0, o1, o2, o3, dp0, dp1, dp2, dp3,
               b0, b1, b2, b3, w1a, w2a, w1b, w2b,
               g0, g1, g2, g3, wsum):
    i = pl.program_id(0)
    gs = []
    for o, dp, b in ((o0, dp0, b0), (o1, dp1, b1), (o2, dp2, b2), (o3, dp3, b3)):
        den = jnp.maximum(dp[0] + dp[1], 1e-9)
        x = o[...]
        cols = []
        for hh in range(H):
            xh = x[:, hh * DH:(hh + 1) * DH] / den[:, hh:hh + 1]
            cols.append(xh)
        x = jnp.concatenate(cols, axis=1) + b[...]
        x = jnp.where(x > 0.0, x, jnp.exp(x) - 1.0)
        gs.append(x)
    g0[...], g1[...], g2[...], g3[...] = gs

    parts = []
    for (ga, gb, w1, w2) in ((gs[0], gs[1], w1a, w2a), (gs[2], gs[3], w1b, w2b)):
        for z in (ga, gb):
            t = jnp.tanh(jnp.dot(z, w1[...], preferred_element_type=jnp.float32))
            s = jnp.sum(jnp.dot(t, w2[...], preferred_element_type=jnp.float32))
            parts.append(s)
    rid = lax.broadcasted_iota(jnp.int32, (8, 128), 0)
    contrib = jnp.zeros((8, 128), jnp.float32)
    for m, s in enumerate(parts):
        contrib = contrib + jnp.where(rid == m, s, 0.0)

    @pl.when(i == 0)
    def _init():
        wsum[...] = contrib

    @pl.when(i > 0)
    def _acc():
        wsum[...] = wsum[...] + contrib


def _k4(outs, dens, bs, w1a, w2a, w1b, w2b):
    BN = 1000
    grid = (N // BN,)
    row = pl.BlockSpec((BN, H * DH), lambda i: (i, 0))
    dsp = pl.BlockSpec((NSC, BN, EL), lambda i: (0, i, 0))
    bsp = pl.BlockSpec((1, H * DH), lambda i: (0, 0))
    wsp1 = pl.BlockSpec((H * DH, 128), lambda i: (0, 0))
    wsp2 = pl.BlockSpec((128, 1), lambda i: (0, 0))
    ssp = pl.BlockSpec((8, 128), lambda i: (0, 0))
    out_shape = ([jax.ShapeDtypeStruct((N, H * DH), jnp.float32)] * 4
                 + [jax.ShapeDtypeStruct((8, 128), jnp.float32)])
    return pl.pallas_call(
        _post_body,
        grid=grid,
        in_specs=[row] * 4 + [dsp] * 4 + [bsp] * 4 + [wsp1, wsp2, wsp1, wsp2],
        out_specs=[row] * 4 + [ssp],
        out_shape=out_shape,
    )(*outs, *dens, *bs, w1a, w2a, w1b, w2b)


def _final_body(g0, g1, g2, g3, wsum, h1w, h1b, h2w, h2b,
                qw, qb, kw, kb, vw, vb, ow, ob,
                drug_o, dis_o):
    def beta(sa, sb):
        wa = sa / jnp.float32(N)
        wb = sb / jnp.float32(N)
        m = jnp.maximum(wa, wb)
        ea = jnp.exp(wa - m)
        eb = jnp.exp(wb - m)
        return ea / (ea + eb), eb / (ea + eb)

    b10, b11 = beta(wsum[0, 0], wsum[1, 0])
    b20, b21 = beta(wsum[2, 0], wsum[3, 0])
    hs1 = b10 * g0[...] + b11 * g1[...]
    hs2 = b20 * g2[...] + b21 * g3[...]
    h1 = jnp.dot(hs1, h1w[...], preferred_element_type=jnp.float32) + h1b[...]
    h2 = jnp.dot(hs2, h2w[...], preferred_element_type=jnp.float32) + h2b[...]

    scale = jnp.sqrt(jnp.float32(OUT))

    def att(q, kk, v):
        Q = jnp.dot(q, qw[...], preferred_element_type=jnp.float32) + qb[...]
        K = jnp.dot(kk, kw[...], preferred_element_type=jnp.float32) + kb[...]
        V = jnp.dot(v, vw[...], preferred_element_type=jnp.float32) + vb[...]
        en = Q[:, :, None] * K[:, None, :] / scale
        en = en - jnp.max(en, axis=-1, keepdims=True)
        ex = jnp.exp(en)
        attn = ex / jnp.sum(ex, axis=-1, keepdims=True)
        wm = jnp.sum(attn * V[:, None, :], axis=-1)
        return jnp.dot(wm, ow[...], preferred_element_type=jnp.float32) + ob[...]

    d = att(h1, h2, h2)
    drug_o[...] = d
    dis_o[...] = att(h2, d, d)


def _k5(gs, wsum, p):
    BN = 400
    grid = (N // BN,)
    row = pl.BlockSpec((BN, H * DH), lambda i: (i, 0))
    ssp = pl.BlockSpec((8, 128), lambda i: (0, 0))
    hw = pl.BlockSpec((H * DH, OUT), lambda i: (0, 0))
    ob = pl.BlockSpec((1, OUT), lambda i: (0, 0))
    aw = pl.BlockSpec((OUT, OUT), lambda i: (0, 0))
    orow = pl.BlockSpec((BN, OUT), lambda i: (i, 0))
    out_shape = [jax.ShapeDtypeStruct((N, OUT), jnp.float32)] * 2
    return pl.pallas_call(
        _final_body,
        grid=grid,
        in_specs=[row] * 4 + [ssp, hw, ob, hw, ob] + [aw, ob] * 4,
        out_specs=[orow, orow],
        out_shape=out_shape,
    )(*gs, wsum,
      p['han1_W'].T, p['han1_b'].reshape(1, OUT),
      p['han2_W'].T, p['han2_b'].reshape(1, OUT),
      p['att_q_W'].T, p['att_q_b'].reshape(1, OUT),
      p['att_k_W'].T, p['att_k_b'].reshape(1, OUT),
      p['att_v_W'].T, p['att_v_b'].reshape(1, OUT),
      p['att_o_W'].T, p['att_o_b'].reshape(1, OUT))


def _alz(al):
    a = (jnp.eye(H, dtype=jnp.float32)[:, None, :] * al[:, :, None]).reshape(H * DH, H)
    return jnp.pad(a, ((0, 0), (0, EL - H)))


def kernel(features_drug, features_dis, params, edge_d0, edge_d1, edge_s0, edge_s1, type_mask):
    p = params
    gat_keys = ['gat10', 'gat11', 'gat20', 'gat21']
    edges = [edge_d0, edge_d1, edge_s0, edge_s1]

    ws = [p[f'{g}_W'].T for g in gat_keys]
    alzs = [_alz(p[f'{g}_al']) for g in gat_keys]
    arzs = [_alz(p[f'{g}_ar']) for g in gat_keys]

    outs1 = _k1(features_drug, features_dis,
                p['fc0_W'].T, p['fc1_W'].T, ws, alzs, arzs)
    feats, els, ers = outs1[0:4], outs1[4:8], outs1[8:12]

    z4 = jnp.zeros((N, EL), jnp.float32)

    gouts, dens = [], []
    for g in range(4):
        src = edges[g][0]
        dst = edges[g][1]
        ex, den = _edge_a(src, dst, els[g], ers[g], z4)
        exT = _ex_transpose(ex)
        featv = feats[g].reshape(2 * H * N, EL)
        oraw = _edge_b(src, dst, exT, featv, z4)
        gouts.append(oraw.reshape(N, H * DH))
        dens.append(den)

    bs = [p[f'{g}_b'].reshape(1, H * DH) for g in gat_keys]
    k4 = _k4(gouts, dens, bs,
             p['sem1_W1'].T, p['sem1_W2'].T, p['sem2_W1'].T, p['sem2_W2'].T)
    gs, wsum = k4[0:4], k4[4]

    drug_o, dis_o = _k5(gs, wsum, p)
    return drug_o, dis_o


# speedup vs baseline: 46.8431x; 1.4490x over previous
import functools

import jax
import jax.numpy as jnp
from jax import lax
from jax.experimental import pallas as pl
from jax.experimental.pallas import tpu as pltpu
from jax.experimental.pallas import tpu_sc as plsc

N = 50000
E = 800000
D_FEAT = 128
HID = 32
H = 4
DH = 32
OUT = 32
EL = 16

NT = 16
NSC = 2
ROWS_PER_TILE = N // NT
CA = 1000
CB = 2000


def _proj_body(fd, fs, fc0w, fc1w, w0, w1, w2, w3,
               alz0, alz1, alz2, alz3, arz0, arz1, arz2, arz3,
               feat0, feat1, feat2, feat3,
               el0, el1, el2, el3, er0, er1, er2, er3):
    tf0 = jnp.dot(fd[...], fc0w[...], preferred_element_type=jnp.float32)
    tf1 = jnp.dot(fs[...], fc1w[...], preferred_element_type=jnp.float32)
    for tf, w, alz, arz, fo, elo, ero in (
            (tf0, w0, alz0, arz0, feat0, el0, er0),
            (tf0, w1, alz1, arz1, feat1, el1, er1),
            (tf1, w2, alz2, arz2, feat2, el2, er2),
            (tf1, w3, alz3, arz3, feat3, el3, er3)):
        f = jnp.dot(tf, w[...], preferred_element_type=jnp.float32)
        fo[...] = f
        elo[...] = jnp.dot(f, alz[...], preferred_element_type=jnp.float32)
        ero[...] = jnp.dot(f, arz[...], preferred_element_type=jnp.float32)


def _k1(fd, fs, fc0w, fc1w, ws, alzs, arzs):
    BN = 1000
    grid = (N // BN,)
    row = pl.BlockSpec((BN, D_FEAT), lambda i: (i, 0))
    w32 = pl.BlockSpec((D_FEAT, HID), lambda i: (0, 0))
    w128 = pl.BlockSpec((HID, H * DH), lambda i: (0, 0))
    wz = pl.BlockSpec((H * DH, EL), lambda i: (0, 0))
    orow = pl.BlockSpec((BN, H * DH), lambda i: (i, 0))
    oel = pl.BlockSpec((BN, EL), lambda i: (i, 0))
    out_shape = ([jax.ShapeDtypeStruct((N, H * DH), jnp.float32)] * 4
                 + [jax.ShapeDtypeStruct((N, EL), jnp.float32)] * 8)
    return pl.pallas_call(
        _proj_body,
        grid=grid,
        in_specs=[row, row, w32, w32] + [w128] * 4 + [wz] * 8,
        out_specs=[orow] * 4 + [oel] * 8,
        out_shape=out_shape,
    )(fd, fs, fc0w, fc1w, *ws, *alzs, *arzs)


def _edge_a(src, dst, el, er, z4):
    mesh = plsc.VectorSubcoreMesh(core_axis_name="c", subcore_axis_name="s")
    epsc = E // NSC
    nchunk = epsc // CA // NT

    @functools.partial(
        pl.kernel,
        out_type=[jax.ShapeDtypeStruct((E, EL), jnp.float32),
                  jax.ShapeDtypeStruct((NSC, N, EL), jnp.float32)],
        mesh=mesh,
        compiler_params=pltpu.CompilerParams(use_tc_tiling_on_sc=False, needs_layout_passes=False),
        scratch_types=[
            pltpu.VMEM((CA,), jnp.int32),
            pltpu.VMEM((CA,), jnp.int32),
            pltpu.VMEM((CA, EL), jnp.float32),
            pltpu.VMEM((CA, EL), jnp.float32),
            pltpu.VMEM((CA, EL), jnp.float32),
            pltpu.VMEM_SHARED((N, EL), jnp.float32),
        ],
    )
    def k(src_h, dst_h, el_h, er_h, z4_h, ex_h, den_h,
          idxs, idxd, elr, err, exb, den_sh):
        cid = lax.axis_index("c")
        sid = lax.axis_index("s")

        @pl.when(sid == 0)
        def _zero():
            pltpu.sync_copy(z4_h, den_sh)

        plsc.subcore_barrier()

        def chunk(kk, _):
            base = cid * epsc + (kk * NT + sid) * CA
            pltpu.sync_copy(src_h.at[pl.ds(base, CA)], idxs)
            pltpu.sync_copy(dst_h.at[pl.ds(base, CA)], idxd)
            pltpu.sync_copy(el_h.at[idxs], elr)
            pltpu.sync_copy(er_h.at[idxd], err)

            @plsc.parallel_loop(0, CA, step=1, unroll=8)
            def cb(e):
                a = elr[e, :] + err[e, :]
                a = jnp.where(a >= 0.0, a, 0.2 * a)
                exb[e, :] = jnp.exp(a)
            pltpu.sync_copy(exb, ex_h.at[pl.ds(base, CA)])
            pltpu.sync_copy(exb, den_sh.at[idxd], add=True)
            return 0

        lax.fori_loop(0, nchunk, chunk, 0)
        plsc.subcore_barrier()

        @pl.when(sid == 0)
        def _out():
            pltpu.sync_copy(den_sh, den_h.at[cid])

    return k(src, dst, el, er, z4)


def _ext_body(ex_ref, out_ref):
    x = ex_ref[...]
    out_ref[...] = x.T[0:8, :]


def _ex_transpose(ex):
    BN = 6400
    grid = (E // BN,)
    return pl.pallas_call(
        _ext_body,
        grid=grid,
        in_specs=[pl.BlockSpec((BN, EL), lambda i: (i, 0))],
        out_specs=pl.BlockSpec((8, BN), lambda i: (0, i)),
        out_shape=jax.ShapeDtypeStruct((8, E), jnp.float32),
    )(ex)


def _edge_b(src, dst, ex, featv, z4):
    mesh = plsc.VectorSubcoreMesh(core_axis_name="c", subcore_axis_name="s")
    nchunk = E // CB // NT

    @functools.partial(
        pl.kernel,
        out_type=jax.ShapeDtypeStruct((N, H, 2, EL), jnp.float32),
        mesh=mesh,
        compiler_params=pltpu.CompilerParams(use_tc_tiling_on_sc=False, needs_layout_passes=False),
        scratch_types=[
            pltpu.VMEM((CB,), jnp.int32),
            pltpu.VMEM((CB,), jnp.int32),
            pltpu.VMEM((CB,), jnp.int32),
            pltpu.VMEM((CB,), jnp.float32),
            pltpu.VMEM((CB, EL), jnp.float32),
            pltpu.VMEM_SHARED((N, EL), jnp.float32),
        ],
    )
    def k(src_h, dst_h, ex_h, featv_h, z4_h, out_h,
          idxs, idxd, idxg, exr, rows, acc_sh):
        cid = lax.axis_index("c")
        sid = lax.axis_index("s")

        for hh in range(2):
            h = cid * 2 + hh
            for half in range(2):
                r0 = sid * ROWS_PER_TILE
                pltpu.sync_copy(z4_h.at[pl.ds(r0, ROWS_PER_TILE)],
                                acc_sh.at[pl.ds(r0, ROWS_PER_TILE)])
                plsc.subcore_barrier()

                def chunk(kk, _):
                    base = (kk * NT + sid) * CB
                    pltpu.sync_copy(src_h.at[pl.ds(base, CB)], idxs)
                    pltpu.sync_copy(dst_h.at[pl.ds(base, CB)], idxd)
                    pltpu.sync_copy(ex_h.at[h, pl.ds(base, CB)], exr)
                    sub = h * 2 + half

                    @plsc.parallel_loop(0, CB, step=16, unroll=4)
                    def ib(i):
                        v = idxs[pl.ds(i, 16)]
                        idxg[pl.ds(i, 16)] = v * (2 * H) + sub

                    pltpu.sync_copy(featv_h.at[idxg], rows)

                    @plsc.parallel_loop(0, CB, step=16, unroll=2)
                    def mul(e0):
                        xv16 = exr[pl.ds(e0, 16)]
                        for r in range(16):
                            xv = jnp.full((16,), xv16[r])
                            rows[e0 + r, :] = rows[e0 + r, :] * xv
                    pltpu.sync_copy(rows, acc_sh.at[idxd], add=True)
                    return 0

                lax.fori_loop(0, nchunk, chunk, 0)
                plsc.subcore_barrier()
                pltpu.sync_copy(acc_sh.at[pl.ds(r0, ROWS_PER_TILE)],
                                out_h.at[pl.ds(r0, ROWS_PER_TILE), h, half])
                plsc.subcore_barrier()

    return k(src, dst, ex, featv, z4)


def _post_body(o0, o1, o2, o3, dp0, dp1, dp2, dp3,
               b0, b1, b2, b3, w1a, w2a, w1b, w2b,
               g0, g1, g2, g3, wsum):
    i = pl.program_id(0)
    gs = []
    for o, dp, b in ((o0, dp0, b0), (o1, dp1, b1), (o2, dp2, b2), (o3, dp3, b3)):
        den = jnp.maximum(dp[0] + dp[1], 1e-9)
        x = o[...]
        cols = []
        for hh in range(H):
            xh = x[:, hh * DH:(hh + 1) * DH] / den[:, hh:hh + 1]
            cols.append(xh)
        x = jnp.concatenate(cols, axis=1) + b[...]
        x = jnp.where(x > 0.0, x, jnp.exp(x) - 1.0)
        gs.append(x)
    g0[...], g1[...], g2[...], g3[...] = gs

    parts = []
    for (ga, gb, w1, w2) in ((gs[0], gs[1], w1a, w2a), (gs[2], gs[3], w1b, w2b)):
        for z in (ga, gb):
            t = jnp.tanh(jnp.dot(z, w1[...], preferred_element_type=jnp.float32))
            s = jnp.sum(jnp.dot(t, w2[...], preferred_element_type=jnp.float32))
            parts.append(s)
    rid = lax.broadcasted_iota(jnp.int32, (8, 128), 0)
    contrib = jnp.zeros((8, 128), jnp.float32)
    for m, s in enumerate(parts):
        contrib = contrib + jnp.where(rid == m, s, 0.0)

    @pl.when(i == 0)
    def _init():
        wsum[...] = contrib

    @pl.when(i > 0)
    def _acc():
        wsum[...] = wsum[...] + contrib


def _k4(outs, dens, bs, w1a, w2a, w1b, w2b):
    BN = 1000
    grid = (N // BN,)
    row = pl.BlockSpec((BN, H * DH), lambda i: (i, 0))
    dsp = pl.BlockSpec((NSC, BN, EL), lambda i: (0, i, 0))
    bsp = pl.BlockSpec((1, H * DH), lambda i: (0, 0))
    wsp1 = pl.BlockSpec((H * DH, 128), lambda i: (0, 0))
    wsp2 = pl.BlockSpec((128, 1), lambda i: (0, 0))
    ssp = pl.BlockSpec((8, 128), lambda i: (0, 0))
    out_shape = ([jax.ShapeDtypeStruct((N, H * DH), jnp.float32)] * 4
                 + [jax.ShapeDtypeStruct((8, 128), jnp.float32)])
    return pl.pallas_call(
        _post_body,
        grid=grid,
        in_specs=[row] * 4 + [dsp] * 4 + [bsp] * 4 + [wsp1, wsp2, wsp1, wsp2],
        out_specs=[row] * 4 + [ssp],
        out_shape=out_shape,
    )(*outs, *dens, *bs, w1a, w2a, w1b, w2b)


def _final_body(g0, g1, g2, g3, wsum, h1w, h1b, h2w, h2b,
                qw, qb, kw, kb, vw, vb, ow, ob,
                drug_o, dis_o):
    def beta(sa, sb):
        wa = sa / jnp.float32(N)
        wb = sb / jnp.float32(N)
        m = jnp.maximum(wa, wb)
        ea = jnp.exp(wa - m)
        eb = jnp.exp(wb - m)
        return ea / (ea + eb), eb / (ea + eb)

    b10, b11 = beta(wsum[0, 0], wsum[1, 0])
    b20, b21 = beta(wsum[2, 0], wsum[3, 0])
    hs1 = b10 * g0[...] + b11 * g1[...]
    hs2 = b20 * g2[...] + b21 * g3[...]
    h1 = jnp.dot(hs1, h1w[...], preferred_element_type=jnp.float32) + h1b[...]
    h2 = jnp.dot(hs2, h2w[...], preferred_element_type=jnp.float32) + h2b[...]

    scale = jnp.sqrt(jnp.float32(OUT))

    def att(q, kk, v):
        Q = jnp.dot(q, qw[...], preferred_element_type=jnp.float32) + qb[...]
        K = jnp.dot(kk, kw[...], preferred_element_type=jnp.float32) + kb[...]
        V = jnp.dot(v, vw[...], preferred_element_type=jnp.float32) + vb[...]
        en = Q[:, :, None] * K[:, None, :] / scale
        en = en - jnp.max(en, axis=-1, keepdims=True)
        ex = jnp.exp(en)
        attn = ex / jnp.sum(ex, axis=-1, keepdims=True)
        wm = jnp.sum(attn * V[:, None, :], axis=-1)
        return jnp.dot(wm, ow[...], preferred_element_type=jnp.float32) + ob[...]

    d = att(h1, h2, h2)
    drug_o[...] = d
    dis_o[...] = att(h2, d, d)


def _k5(gs, wsum, p):
    BN = 400
    grid = (N // BN,)
    row = pl.BlockSpec((BN, H * DH), lambda i: (i, 0))
    ssp = pl.BlockSpec((8, 128), lambda i: (0, 0))
    hw = pl.BlockSpec((H * DH, OUT), lambda i: (0, 0))
    ob = pl.BlockSpec((1, OUT), lambda i: (0, 0))
    aw = pl.BlockSpec((OUT, OUT), lambda i: (0, 0))
    orow = pl.BlockSpec((BN, OUT), lambda i: (i, 0))
    out_shape = [jax.ShapeDtypeStruct((N, OUT), jnp.float32)] * 2
    return pl.pallas_call(
        _final_body,
        grid=grid,
        in_specs=[row] * 4 + [ssp, hw, ob, hw, ob] + [aw, ob] * 4,
        out_specs=[orow, orow],
        out_shape=out_shape,
    )(*gs, wsum,
      p['han1_W'].T, p['han1_b'].reshape(1, OUT),
      p['han2_W'].T, p['han2_b'].reshape(1, OUT),
      p['att_q_W'].T, p['att_q_b'].reshape(1, OUT),
      p['att_k_W'].T, p['att_k_b'].reshape(1, OUT),
      p['att_v_W'].T, p['att_v_b'].reshape(1, OUT),
      p['att_o_W'].T, p['att_o_b'].reshape(1, OUT))


def _alz(al):
    a = (jnp.eye(H, dtype=jnp.float32)[:, None, :] * al[:, :, None]).reshape(H * DH, H)
    return jnp.pad(a, ((0, 0), (0, EL - H)))


def kernel(features_drug, features_dis, params, edge_d0, edge_d1, edge_s0, edge_s1, type_mask):
    p = params
    gat_keys = ['gat10', 'gat11', 'gat20', 'gat21']
    edges = [edge_d0, edge_d1, edge_s0, edge_s1]

    ws = [p[f'{g}_W'].T for g in gat_keys]
    alzs = [_alz(p[f'{g}_al']) for g in gat_keys]
    arzs = [_alz(p[f'{g}_ar']) for g in gat_keys]

    outs1 = _k1(features_drug, features_dis,
                p['fc0_W'].T, p['fc1_W'].T, ws, alzs, arzs)
    feats, els, ers = outs1[0:4], outs1[4:8], outs1[8:12]

    z4 = jnp.zeros((N, EL), jnp.float32)

    gouts, dens = [], []
    for g in range(4):
        src = edges[g][0]
        dst = edges[g][1]
        ex, den = _edge_a(src, dst, els[g], ers[g], z4)
        exT = _ex_transpose(ex)
        featv = feats[g].reshape(2 * H * N, EL)
        oraw = _edge_b(src, dst, exT, featv, z4)
        gouts.append(oraw.reshape(N, H * DH))
        dens.append(den)

    bs = [p[f'{g}_b'].reshape(1, H * DH) for g in gat_keys]
    k4 = _k4(gouts, dens, bs,
             p['sem1_W1'].T, p['sem1_W2'].T, p['sem2_W1'].T, p['sem2_W2'].T)
    gs, wsum = k4[0:4], k4[4]

    drug_o, dis_o = _k5(gs, wsum, p)
    return drug_o, dis_o
